# Initial kernel scaffold; baseline (speedup 1.0000x reference)
import jax
import jax.numpy as jnp
from jax.experimental import pallas as pl


def kernel(x, edge_index, W1, a_src1, a_dst1, b1, W2, a_src2, a_dst2, b2, W3, a_src3, a_dst3, b3):
    raise NotImplementedError("write your pallas kernel here")


# speedup vs baseline: 1.1292x; 1.1292x over previous
import functools

import jax
import jax.numpy as jnp
from jax.experimental import pallas as pl
from jax.experimental.pallas import tpu as pltpu

_N = 50000
_RB = 2000


def _dense_body(x_ref, w_ref, a_ref, h_ref, aa_ref):
    h = jnp.dot(x_ref[...], w_ref[...], preferred_element_type=jnp.float32)
    h_ref[...] = h
    aa_ref[...] = jnp.dot(h, a_ref[...], preferred_element_type=jnp.float32)


def _dense(x, W, A):
    n, k = x.shape
    c = W.shape[1]
    grid = n // _RB
    return pl.pallas_call(
        _dense_body,
        grid=(grid,),
        in_specs=[
            pl.BlockSpec((_RB, k), lambda i: (i, 0)),
            pl.BlockSpec((k, c), lambda i: (0, 0)),
            pl.BlockSpec((c, 8), lambda i: (0, 0)),
        ],
        out_specs=[
            pl.BlockSpec((_RB, c), lambda i: (i, 0)),
            pl.BlockSpec((_RB, 8), lambda i: (i, 0)),
        ],
        out_shape=[
            jax.ShapeDtypeStruct((n, c), jnp.float32),
            jax.ShapeDtypeStruct((n, 8), jnp.float32),
        ],
    )(x, W, A)


def _edge_phase(h, aa, src, dst, heads, out_ch):
    n = h.shape[0]
    hr = h.reshape(n, heads, out_ch)
    alpha_src = aa[:, :heads]
    alpha_dst = aa[:, 4:4 + heads]
    e = alpha_src[src] + alpha_dst[dst]
    e = jax.nn.leaky_relu(e, negative_slope=0.2)
    g = jnp.exp(e)
    denom = jax.ops.segment_sum(g, dst, num_segments=n)
    msg = hr[src] * g[:, :, None]
    acc = jax.ops.segment_sum(msg, dst, num_segments=n)
    out = acc / (denom[:, :, None] + 1e-16)
    return out.reshape(n, heads * out_ch)


def _proj_mat(a_src, a_dst, heads, out_ch, c):
    A = jnp.zeros((c, 8), jnp.float32)
    hr = jnp.arange(c) // out_ch
    cr = jnp.arange(c) % out_ch
    A = A.at[jnp.arange(c), hr].set(a_src[hr, cr])
    A = A.at[jnp.arange(c), 4 + hr].set(a_dst[hr, cr])
    return A


def kernel(x, edge_index, W1, a_src1, a_dst1, b1, W2, a_src2, a_dst2, b2,
           W3, a_src3, a_dst3, b3):
    n = x.shape[0]
    loop = jnp.arange(n, dtype=edge_index.dtype)
    src = jnp.concatenate([edge_index[0], loop])
    dst = jnp.concatenate([edge_index[1], loop])

    xp = jnp.pad(x, ((0, 0), (0, 1)))
    W1p = jnp.pad(W1, ((0, 1), (0, 0)))
    A1 = _proj_mat(a_src1, a_dst1, 4, 32, 128)
    h, aa = _dense(xp, W1p, A1)
    o = _edge_phase(h, aa, src, dst, 4, 32) + b1
    o = jax.nn.elu(o)

    A2 = _proj_mat(a_src2, a_dst2, 4, 16, 64)
    h, aa = _dense(o, W2, A2)
    o = _edge_phase(h, aa, src, dst, 4, 16) + b2
    o = jax.nn.elu(o)

    W3p = jnp.pad(W3, ((0, 0), (0, 127)))
    A3 = jnp.zeros((128, 8), jnp.float32).at[0, 0].set(a_src3[0, 0]).at[0, 4].set(a_dst3[0, 0])
    h, aa = _dense(o, W3p, A3)
    o = _edge_phase(h[:, :128], aa, src, dst, 1, 128)[:, 0] + b3[0]
    return jax.nn.sigmoid(o) * 100.0


# speedup vs baseline: 11.2506x; 9.9630x over previous
import functools

import jax
import jax.numpy as jnp
from jax import lax
from jax.experimental import pallas as pl
from jax.experimental.pallas import tpu as pltpu
from jax.experimental.pallas import tpu_sc as plsc

_N = 50000
_NP = 51200
_CHUNK = 12800
_ACC_ROWS = 12816
_E = 800000
_EP = 851968
_RB = 1600


def _tc_prep1(xp, W1p, A1):
    def body(x_ref, w_ref, a_ref, t_ref, ad_ref):
        h = jnp.dot(x_ref[...], w_ref[...], preferred_element_type=jnp.float32)
        aa = jnp.dot(h, a_ref[...], preferred_element_type=jnp.float32)
        t_ref[:, 0:128] = h
        t_ref[:, 128:136] = aa
        t_ref[:, 136:144] = jnp.zeros((_RB, 8), jnp.float32)
        ad_ref[:, 0:4] = aa[:, 4:8]
        ad_ref[:, 4:16] = jnp.zeros((_RB, 12), jnp.float32)

    return pl.pallas_call(
        body,
        grid=(_NP // _RB,),
        in_specs=[
            pl.BlockSpec((_RB, 8), lambda i: (i, 0)),
            pl.BlockSpec((8, 128), lambda i: (0, 0)),
            pl.BlockSpec((128, 8), lambda i: (0, 0)),
        ],
        out_specs=[
            pl.BlockSpec((_RB, 144), lambda i: (i, 0)),
            pl.BlockSpec((_RB, 16), lambda i: (i, 0)),
        ],
        out_shape=[
            jax.ShapeDtypeStruct((_NP, 144), jnp.float32),
            jax.ShapeDtypeStruct((_NP, 16), jnp.float32),
        ],
    )(xp, W1p, A1)


def _tc_mid(OUT, E, b, W, A, C_in, C_out):
    RW_in = C_in + 16
    RW_out = C_out + 16

    def body(o_ref, e_ref, b_ref, w_ref, a_ref, t_ref, ad_ref):
        den = jnp.dot(o_ref[:, 0:4], e_ref[...],
                      preferred_element_type=jnp.float32)
        o = o_ref[:, 16:RW_in] / (den + 1e-16) + b_ref[...]
        o = jnp.where(o > 0, o, (jnp.exp(o) - 1.0))
        h = jnp.dot(o, w_ref[...], preferred_element_type=jnp.float32)
        aa = jnp.dot(h, a_ref[...], preferred_element_type=jnp.float32)
        t_ref[:, 0:C_out] = h
        t_ref[:, C_out:C_out + 8] = aa
        t_ref[:, C_out + 8:RW_out] = jnp.zeros((_RB, 8), jnp.float32)
        ad_ref[:, 0:4] = aa[:, 4:8]
        ad_ref[:, 4:16] = jnp.zeros((_RB, 12), jnp.float32)

    return pl.pallas_call(
        body,
        grid=(_NP // _RB,),
        in_specs=[
            pl.BlockSpec((_RB, RW_in), lambda i: (i, 0)),
            pl.BlockSpec((4, C_in), lambda i: (0, 0)),
            pl.BlockSpec((1, C_in), lambda i: (0, 0)),
            pl.BlockSpec((C_in, C_out), lambda i: (0, 0)),
            pl.BlockSpec((C_out, 8), lambda i: (0, 0)),
        ],
        out_specs=[
            pl.BlockSpec((_RB, RW_out), lambda i: (i, 0)),
            pl.BlockSpec((_RB, 16), lambda i: (i, 0)),
        ],
        out_shape=[
            jax.ShapeDtypeStruct((_NP, RW_out), jnp.float32),
            jax.ShapeDtypeStruct((_NP, 16), jnp.float32),
        ],
    )(OUT, E, b, W, A)


def _tc_fin2(OUT, E, b, W3eff):
    def body(o_ref, e_ref, b_ref, w_ref, t_ref):
        den = jnp.dot(o_ref[:, 0:4], e_ref[...],
                      preferred_element_type=jnp.float32)
        o = o_ref[:, 16:80] / (den + 1e-16) + b_ref[...]
        o = jnp.where(o > 0, o, (jnp.exp(o) - 1.0))
        t_ref[...] = jnp.dot(o, w_ref[...], preferred_element_type=jnp.float32)

    return pl.pallas_call(
        body,
        grid=(_NP // _RB,),
        in_specs=[
            pl.BlockSpec((_RB, 80), lambda i: (i, 0)),
            pl.BlockSpec((4, 64), lambda i: (0, 0)),
            pl.BlockSpec((1, 64), lambda i: (0, 0)),
            pl.BlockSpec((64, 16), lambda i: (0, 0)),
        ],
        out_specs=pl.BlockSpec((_RB, 16), lambda i: (i, 0)),
        out_shape=jax.ShapeDtypeStruct((_NP, 16), jnp.float32),
    )(OUT, E, b, W3eff)


def _tc_final(O0, O1, b3s):
    RB = 2048

    def body(o0_ref, o1_ref, b_ref, z_ref):
        acc = o0_ref[...] + o1_ref[...]
        z = acc[:, 1:2] / (acc[:, 0:1] + 1e-16) + b_ref[...]
        z_ref[...] = jax.nn.sigmoid(z) * 100.0

    return pl.pallas_call(
        body,
        grid=(_NP // RB,),
        in_specs=[
            pl.BlockSpec((RB, 16), lambda i: (i, 0)),
            pl.BlockSpec((RB, 16), lambda i: (i, 0)),
            pl.BlockSpec((1, 1), lambda i: (0, 0)),
        ],
        out_specs=pl.BlockSpec((RB, 1), lambda i: (i, 0)),
        out_shape=jax.ShapeDtypeStruct((_NP, 1), jnp.float32),
    )(O0, O1, b3s)


_B = 128


def _sc_edge(C, chunk):
    RW = C + 16
    acc_rows = chunk + 16
    zrows = acc_rows // 16
    frows = chunk // 16
    cps = _NP // chunk // 2
    blocks = _EP // 16 // _B
    mesh = plsc.VectorSubcoreMesh(core_axis_name="c", subcore_axis_name="s")

    @functools.partial(
        pl.kernel,
        out_type=jax.ShapeDtypeStruct((_NP, RW), jnp.float32),
        mesh=mesh,
        compiler_params=pltpu.CompilerParams(
            use_tc_tiling_on_sc=False, needs_layout_passes=False),
        scratch_types=[
            pltpu.VMEM((_B,), jnp.int32),
            pltpu.VMEM((_B,), jnp.int32),
            pltpu.VMEM((_B,), jnp.int32),
            pltpu.VMEM((_B, 16), jnp.float32),
            pltpu.VMEM((_B, RW), jnp.float32),
            pltpu.VMEM((_B, RW), jnp.float32),
            pltpu.VMEM((4 * _B,), jnp.float32),
            pltpu.VMEM_SHARED((acc_rows, RW), jnp.float32),
            pltpu.SemaphoreType.DMA,
            pltpu.SemaphoreType.DMA,
        ],
    )
    def kern(T, AD, SRC, DST, OUT, s_idx, d_idx, l_idx, adbuf, rowbuf, bbuf,
             gbuf, acc, sem1, sem2):
        cid = lax.axis_index("c")
        sid = lax.axis_index("s")
        lanes = lax.iota(jnp.int32, 16)
        zv = jnp.zeros((16,), jnp.float32)
        ebase = sid * (_EP // 16)

        def zero_bbuf():
            def zrow(i, carry):
                r = i // (RW // 16)
                co = (i % (RW // 16)) * 16
                bbuf[r, pl.ds(co, 16)] = zv
                return carry
            lax.fori_loop(0, _B * (RW // 16), zrow, 0)

        def do_block(b, lo):
            off = ebase + b * _B
            pltpu.sync_copy(SRC.at[pl.ds(off, _B)], s_idx)
            pltpu.sync_copy(DST.at[pl.ds(off, _B)], d_idx)

            def f_l(v, carry):
                d16 = d_idx[pl.ds(v * 16, 16)]
                l16 = d16 - lo
                ok = (l16 >= 0) & (l16 < chunk)
                l_idx[pl.ds(v * 16, 16)] = jnp.where(ok, l16, chunk)
                return carry
            lax.fori_loop(0, _B // 16, f_l, 0)

            cp1 = pltpu.async_copy(T.at[s_idx], rowbuf, sem1)
            cp2 = pltpu.async_copy(AD.at[d_idx], adbuf, sem2)
            cp1.wait()
            cp2.wait()

            def f_g(v, carry):
                e4 = v * 4 + lanes // 4
                hh = lanes % 4
                as16 = plsc.load_gather(rowbuf, [e4, C + hh])
                ad16 = plsc.load_gather(adbuf, [e4, hh])
                e16 = as16 + ad16
                g16 = jnp.exp(jnp.maximum(e16, 0.2 * e16))
                plsc.store_scatter(gbuf, [e4 * 4 + hh], g16)
                return carry
            lax.fori_loop(0, _B // 4, f_g, 0)

            zi = jnp.zeros((16,), jnp.int32)

            def f_b(e, carry):
                g4 = plsc.load_gather(gbuf, [e * 4 + jnp.minimum(lanes, 3)])
                plsc.store_scatter(bbuf, [zi + e, lanes], g4, mask=lanes < 4)
                for j in range(C // 16):
                    head = (j * 16) // (C // 4)
                    ge = plsc.load_gather(gbuf, [zi + (e * 4 + head)])
                    h16 = rowbuf[e, pl.ds(j * 16, 16)]
                    bbuf[e, pl.ds(16 + j * 16, 16)] = h16 * ge
                return carry
            lax.fori_loop(0, _B, f_b, 0)

            pltpu.sync_copy(bbuf, acc.at[l_idx], add=True)

        for k_ in range(cps):
            lo = (cid * cps + k_) * chunk
            zero_bbuf()
            for i in range(zrows // _B):
                pltpu.sync_copy(bbuf, acc.at[pl.ds(sid * zrows + i * _B, _B)])
            pltpu.sync_copy(bbuf.at[pl.ds(0, zrows % _B)],
                            acc.at[pl.ds(sid * zrows + (zrows // _B) * _B,
                                         zrows % _B)])
            plsc.subcore_barrier()

            def f_blk(b, carry):
                do_block(b, lo)
                return carry
            lax.fori_loop(0, blocks, f_blk, 0)
            plsc.subcore_barrier()

            pltpu.sync_copy(acc.at[pl.ds(sid * frows, frows)],
                            OUT.at[pl.ds(lo + sid * frows, frows)])
            plsc.subcore_barrier()

    return kern


def _sc_edge3():
    per_tile = _EP // 32
    blocks = per_tile // _B
    mesh = plsc.VectorSubcoreMesh(core_axis_name="c", subcore_axis_name="s")

    @functools.partial(
        pl.kernel,
        out_type=[jax.ShapeDtypeStruct((_NP, 16), jnp.float32),
                  jax.ShapeDtypeStruct((_NP, 16), jnp.float32)],
        mesh=mesh,
        compiler_params=pltpu.CompilerParams(
            use_tc_tiling_on_sc=False, needs_layout_passes=False),
        scratch_types=[
            pltpu.VMEM((_B,), jnp.int32),
            pltpu.VMEM((_B,), jnp.int32),
            pltpu.VMEM((_B, 16), jnp.float32),
            pltpu.VMEM((_B, 16), jnp.float32),
            pltpu.VMEM((_B, 16), jnp.float32),
            pltpu.VMEM_SHARED((_NP, 16), jnp.float32),
            pltpu.SemaphoreType.DMA,
            pltpu.SemaphoreType.DMA,
        ],
    )
    def kern(T3, SRC, DST, O0, O1, s_idx, d_idx, rowS, rowD, bbuf, acc,
             sem1, sem2):
        cid = lax.axis_index("c")
        sid = lax.axis_index("s")
        lanes = lax.iota(jnp.int32, 16)
        zv = jnp.zeros((16,), jnp.float32)
        zi = jnp.zeros((16,), jnp.int32)
        base = (sid * 2 + cid) * per_tile

        def zrow(i, carry):
            bbuf[i, pl.ds(0, 16)] = zv
            return carry
        lax.fori_loop(0, _B, zrow, 0)

        for i in range(25):
            pltpu.sync_copy(bbuf, acc.at[pl.ds(sid * 3200 + i * _B, _B)])
        plsc.subcore_barrier()

        def do_block(b, carry):
            off = base + b * _B
            pltpu.sync_copy(SRC.at[pl.ds(off, _B)], s_idx)
            pltpu.sync_copy(DST.at[pl.ds(off, _B)], d_idx)
            cp1 = pltpu.async_copy(T3.at[s_idx], rowS, sem1)
            cp2 = pltpu.async_copy(T3.at[d_idx], rowD, sem2)
            cp1.wait()
            cp2.wait()

            def f_g(v, c2):
                r = v * 16 + lanes
                h16 = plsc.load_gather(rowS, [r, zi])
                as16 = plsc.load_gather(rowS, [r, zi + 1])
                ad16 = plsc.load_gather(rowD, [r, zi + 2])
                e16 = as16 + ad16
                g16 = jnp.exp(jnp.maximum(e16, 0.2 * e16))
                plsc.store_scatter(bbuf, [r, zi], g16)
                plsc.store_scatter(bbuf, [r, zi + 1], g16 * h16)
                return c2
            lax.fori_loop(0, _B // 16, f_g, 0)

            pltpu.sync_copy(bbuf, acc.at[d_idx], add=True)
            return carry

        lax.fori_loop(0, blocks, do_block, 0)
        plsc.subcore_barrier()

        @pl.when(cid == 0)
        def _():
            pltpu.sync_copy(acc.at[pl.ds(sid * 3200, 3200)],
                            O0.at[pl.ds(sid * 3200, 3200)])

        @pl.when(cid == 1)
        def _():
            pltpu.sync_copy(acc.at[pl.ds(sid * 3200, 3200)],
                            O1.at[pl.ds(sid * 3200, 3200)])

    return kern


def _proj_mat(a_src, a_dst, heads, out_ch, c):
    A = jnp.zeros((c, 8), jnp.float32)
    hr = jnp.arange(c) // out_ch
    cr = jnp.arange(c) % out_ch
    A = A.at[jnp.arange(c), hr].set(a_src[hr, cr])
    A = A.at[jnp.arange(c), 4 + hr].set(a_dst[hr, cr])
    return A


def _expand_mat(heads, out_ch):
    c = heads * out_ch
    E = jnp.zeros((4, c), jnp.float32)
    return E.at[jnp.arange(c) // out_ch, jnp.arange(c)].set(1.0)


def kernel(x, edge_index, W1, a_src1, a_dst1, b1, W2, a_src2, a_dst2, b2,
           W3, a_src3, a_dst3, b3):
    n = x.shape[0]
    loop = jnp.arange(n, dtype=edge_index.dtype)
    src = jnp.concatenate([edge_index[0], loop])
    dst = jnp.concatenate([edge_index[1], loop])
    srcp = jnp.pad(src, (0, _EP - src.shape[0]))
    dstp = jnp.pad(dst, (0, _EP - dst.shape[0]), constant_values=_NP - 1)

    xp = jnp.pad(x, ((0, _NP - n), (0, 1)))
    W1p = jnp.pad(W1, ((0, 1), (0, 0)))
    A1 = _proj_mat(a_src1, a_dst1, 4, 32, 128)
    A2 = _proj_mat(a_src2, a_dst2, 4, 16, 64)
    E1 = _expand_mat(4, 32)
    E2 = _expand_mat(4, 16)
    row3 = jnp.zeros((1, 16), jnp.float32)
    row3 = row3.at[0, 0].set(1.0).at[0, 1].set(a_src3[0, 0])
    row3 = row3.at[0, 2].set(a_dst3[0, 0])
    W3eff = W3 @ row3

    T1, AD1 = _tc_prep1(xp, W1p, A1)
    OUT1 = _sc_edge(128, 6400)(T1, AD1, srcp, dstp)
    T2, AD2 = _tc_mid(OUT1, E1, b1.reshape(1, 128), W2, A2, 128, 64)
    OUT2 = _sc_edge(64, 12800)(T2, AD2, srcp, dstp)
    T3 = _tc_fin2(OUT2, E2, b2.reshape(1, 64), W3eff)
    O0, O1 = _sc_edge3()(T3, srcp, dstp)
    z = _tc_final(O0, O1, b3.reshape(1, 1))
    return z[:_N, 0]


# speedup vs baseline: 36.0217x; 3.2018x over previous
import functools

import jax
import jax.numpy as jnp
from jax import lax
from jax.experimental import pallas as pl
from jax.experimental.pallas import tpu as pltpu
from jax.experimental.pallas import tpu_sc as plsc

_N = 50000
_NP = 51200
_CHUNK = 12800
_ACC_ROWS = 12816
_E = 800000
_EP = 851968
_RB = 1600


def _tc_prep1(xp, W1p, A1):
    def body(x_ref, w_ref, a_ref, t_ref, ad_ref):
        h = jnp.dot(x_ref[...], w_ref[...], preferred_element_type=jnp.float32)
        aa = jnp.dot(h, a_ref[...], preferred_element_type=jnp.float32)
        t_ref[:, 0:128] = h
        t_ref[:, 128:136] = aa
        t_ref[:, 136:144] = jnp.zeros((_RB, 8), jnp.float32)
        ad_ref[:, 0:4] = aa[:, 4:8]
        ad_ref[:, 4:16] = jnp.zeros((_RB, 12), jnp.float32)

    return pl.pallas_call(
        body,
        grid=(_NP // _RB,),
        in_specs=[
            pl.BlockSpec((_RB, 8), lambda i: (i, 0)),
            pl.BlockSpec((8, 128), lambda i: (0, 0)),
            pl.BlockSpec((128, 8), lambda i: (0, 0)),
        ],
        out_specs=[
            pl.BlockSpec((_RB, 144), lambda i: (i, 0)),
            pl.BlockSpec((_RB, 16), lambda i: (i, 0)),
        ],
        out_shape=[
            jax.ShapeDtypeStruct((_NP, 144), jnp.float32),
            jax.ShapeDtypeStruct((_NP, 16), jnp.float32),
        ],
    )(xp, W1p, A1)


def _tc_mid(OUT, E, b, W, A, C_in, C_out):
    RW_in = C_in + 16
    RW_out = C_out + 16

    def body(o_ref, e_ref, b_ref, w_ref, a_ref, t_ref, ad_ref):
        den = jnp.dot(o_ref[:, 0:4], e_ref[...],
                      preferred_element_type=jnp.float32)
        o = o_ref[:, 16:RW_in] / (den + 1e-16) + b_ref[...]
        o = jnp.where(o > 0, o, (jnp.exp(o) - 1.0))
        h = jnp.dot(o, w_ref[...], preferred_element_type=jnp.float32)
        aa = jnp.dot(h, a_ref[...], preferred_element_type=jnp.float32)
        t_ref[:, 0:C_out] = h
        t_ref[:, C_out:C_out + 8] = aa
        t_ref[:, C_out + 8:RW_out] = jnp.zeros((_RB, 8), jnp.float32)
        ad_ref[:, 0:4] = aa[:, 4:8]
        ad_ref[:, 4:16] = jnp.zeros((_RB, 12), jnp.float32)

    return pl.pallas_call(
        body,
        grid=(_NP // _RB,),
        in_specs=[
            pl.BlockSpec((_RB, RW_in), lambda i: (i, 0)),
            pl.BlockSpec((4, C_in), lambda i: (0, 0)),
            pl.BlockSpec((1, C_in), lambda i: (0, 0)),
            pl.BlockSpec((C_in, C_out), lambda i: (0, 0)),
            pl.BlockSpec((C_out, 8), lambda i: (0, 0)),
        ],
        out_specs=[
            pl.BlockSpec((_RB, RW_out), lambda i: (i, 0)),
            pl.BlockSpec((_RB, 16), lambda i: (i, 0)),
        ],
        out_shape=[
            jax.ShapeDtypeStruct((_NP, RW_out), jnp.float32),
            jax.ShapeDtypeStruct((_NP, 16), jnp.float32),
        ],
    )(OUT, E, b, W, A)


def _tc_fin2(OUT, E, b, W3eff):
    def body(o_ref, e_ref, b_ref, w_ref, t_ref):
        den = jnp.dot(o_ref[:, 0:4], e_ref[...],
                      preferred_element_type=jnp.float32)
        o = o_ref[:, 16:80] / (den + 1e-16) + b_ref[...]
        o = jnp.where(o > 0, o, (jnp.exp(o) - 1.0))
        t_ref[...] = jnp.dot(o, w_ref[...], preferred_element_type=jnp.float32)

    return pl.pallas_call(
        body,
        grid=(_NP // _RB,),
        in_specs=[
            pl.BlockSpec((_RB, 80), lambda i: (i, 0)),
            pl.BlockSpec((4, 64), lambda i: (0, 0)),
            pl.BlockSpec((1, 64), lambda i: (0, 0)),
            pl.BlockSpec((64, 16), lambda i: (0, 0)),
        ],
        out_specs=pl.BlockSpec((_RB, 16), lambda i: (i, 0)),
        out_shape=jax.ShapeDtypeStruct((_NP, 16), jnp.float32),
    )(OUT, E, b, W3eff)


def _tc_final(O0, O1, b3s):
    RB = 2048

    def body(o0_ref, o1_ref, b_ref, z_ref):
        acc = o0_ref[...] + o1_ref[...]
        z = acc[:, 1:2] / (acc[:, 0:1] + 1e-16) + b_ref[...]
        z_ref[...] = jax.nn.sigmoid(z) * 100.0

    return pl.pallas_call(
        body,
        grid=(_NP // RB,),
        in_specs=[
            pl.BlockSpec((RB, 16), lambda i: (i, 0)),
            pl.BlockSpec((RB, 16), lambda i: (i, 0)),
            pl.BlockSpec((1, 1), lambda i: (0, 0)),
        ],
        out_specs=pl.BlockSpec((RB, 1), lambda i: (i, 0)),
        out_shape=jax.ShapeDtypeStruct((_NP, 1), jnp.float32),
    )(O0, O1, b3s)


_B = 128


def _sc_edge(C, chunk):
    RW = C + 16
    acc_rows = chunk + 16
    zrows = acc_rows // 16
    frows = chunk // 16
    cps = _NP // chunk // 2
    blocks = _EP // 16 // _B
    mesh = plsc.VectorSubcoreMesh(core_axis_name="c", subcore_axis_name="s")

    @functools.partial(
        pl.kernel,
        out_type=jax.ShapeDtypeStruct((_NP, RW), jnp.float32),
        mesh=mesh,
        compiler_params=pltpu.CompilerParams(
            use_tc_tiling_on_sc=False, needs_layout_passes=False),
        scratch_types=[
            pltpu.VMEM((_B,), jnp.int32),
            pltpu.VMEM((_B,), jnp.int32),
            pltpu.VMEM((_B,), jnp.int32),
            pltpu.VMEM((2 * _B,), jnp.int32),
            pltpu.VMEM((2 * _B,), jnp.int32),
            pltpu.VMEM((_B,), jnp.int32),
            pltpu.VMEM((_B,), jnp.int32),
            pltpu.VMEM((_B, 16), jnp.float32),
            pltpu.VMEM((_B, RW), jnp.float32),
            pltpu.VMEM((_B, RW), jnp.float32),
            pltpu.VMEM((4 * _B,), jnp.float32),
            pltpu.VMEM_SHARED((acc_rows, RW), jnp.float32),
            pltpu.SemaphoreType.DMA,
            pltpu.SemaphoreType.DMA,
        ],
    )
    def kern(T, AD, SRC, DST, OUT, s_idx, d_idx, l_idx, csrc, cdst, gsrc,
             gdst, adbuf, rowbuf, bbuf, gbuf, acc, sem1, sem2):
        cid = lax.axis_index("c")
        sid = lax.axis_index("s")
        lanes = lax.iota(jnp.int32, 16)
        zv = jnp.zeros((16,), jnp.float32)
        ebase = sid * (_EP // 16)

        def zero_bbuf():
            def zrow(i, carry):
                r = i // (RW // 16)
                co = (i % (RW // 16)) * 16
                bbuf[r, pl.ds(co, 16)] = zv
                return carry
            lax.fori_loop(0, _B * (RW // 16), zrow, 0)

        def process_batch(lo):
            def f_l(v, carry):
                d16 = gdst[pl.ds(v * 16, 16)]
                l16 = jnp.minimum(jnp.maximum(d16 - lo, 0), chunk)
                l_idx[pl.ds(v * 16, 16)] = l16
                return carry
            lax.fori_loop(0, _B // 16, f_l, 0)

            cp1 = pltpu.async_copy(T.at[gsrc], rowbuf, sem1)
            cp2 = pltpu.async_copy(AD.at[gdst], adbuf, sem2)
            cp1.wait()
            cp2.wait()

            def f_g(v, carry):
                e4 = v * 4 + lanes // 4
                hh = lanes % 4
                as16 = plsc.load_gather(rowbuf, [e4, C + hh])
                ad16 = plsc.load_gather(adbuf, [e4, hh])
                e16 = as16 + ad16
                g16 = jnp.exp(jnp.maximum(e16, 0.2 * e16))
                plsc.store_scatter(gbuf, [e4 * 4 + hh], g16)
                return carry
            lax.fori_loop(0, _B // 4, f_g, 0)

            zi = jnp.zeros((16,), jnp.int32)

            def f_b(e, carry):
                g4 = plsc.load_gather(gbuf, [e * 4 + jnp.minimum(lanes, 3)])
                plsc.store_scatter(bbuf, [zi + e, lanes], g4, mask=lanes < 4)
                for j in range(C // 16):
                    head = (j * 16) // (C // 4)
                    ge = plsc.load_gather(gbuf, [zi + (e * 4 + head)])
                    h16 = rowbuf[e, pl.ds(j * 16, 16)]
                    bbuf[e, pl.ds(16 + j * 16, 16)] = h16 * ge
                return carry
            lax.fori_loop(0, _B, f_b, 0)

            pltpu.sync_copy(bbuf, acc.at[l_idx], add=True)

        def scan_block(b, cnt, lo):
            off = ebase + b * _B
            pltpu.sync_copy(SRC.at[pl.ds(off, _B)], s_idx)
            pltpu.sync_copy(DST.at[pl.ds(off, _B)], d_idx)

            def f_app(v, c2):
                d16 = d_idx[pl.ds(v * 16, 16)]
                l16 = d16 - lo
                ok = (l16 >= 0) & (l16 < chunk)
                s16 = s_idx[pl.ds(v * 16, 16)]
                plsc.store_compressed(csrc.at[pl.ds(c2, 16)], s16, mask=ok)
                plsc.store_compressed(cdst.at[pl.ds(c2, 16)], d16, mask=ok)
                return c2 + jnp.sum(ok.astype(jnp.int32))
            cnt = lax.fori_loop(0, _B // 16, f_app, cnt)

            @pl.when(cnt >= _B)
            def _():
                def f_cp(v, c3):
                    gsrc[pl.ds(v * 16, 16)] = csrc[pl.ds(v * 16, 16)]
                    gdst[pl.ds(v * 16, 16)] = cdst[pl.ds(v * 16, 16)]
                    return c3
                lax.fori_loop(0, _B // 16, f_cp, 0)
                process_batch(lo)
                def f_sh(v, c3):
                    csrc[pl.ds(v * 16, 16)] = csrc[pl.ds(_B + v * 16, 16)]
                    cdst[pl.ds(v * 16, 16)] = cdst[pl.ds(_B + v * 16, 16)]
                    return c3
                lax.fori_loop(0, _B // 16, f_sh, 0)
            return jnp.where(cnt >= _B, cnt - _B, cnt)

        def flush_tail(cnt, lo):
            def f_t(v, carry):
                li = v * 16 + lanes
                m = li < cnt
                s16 = csrc[pl.ds(v * 16, 16)]
                d16 = cdst[pl.ds(v * 16, 16)]
                gsrc[pl.ds(v * 16, 16)] = jnp.where(m, s16, 0)
                gdst[pl.ds(v * 16, 16)] = jnp.where(m, d16, _NP - 1)
                return carry
            lax.fori_loop(0, _B // 16, f_t, 0)
            process_batch(lo)

        for k_ in range(cps):
            lo = (cid * cps + k_) * chunk
            zero_bbuf()
            for i in range(zrows // _B):
                pltpu.sync_copy(bbuf, acc.at[pl.ds(sid * zrows + i * _B, _B)])
            pltpu.sync_copy(bbuf.at[pl.ds(0, zrows % _B)],
                            acc.at[pl.ds(sid * zrows + (zrows // _B) * _B,
                                         zrows % _B)])
            plsc.subcore_barrier()

            def f_blk(b, cnt):
                return scan_block(b, cnt, lo)
            cnt = lax.fori_loop(0, blocks, f_blk, jnp.int32(0))
            flush_tail(cnt, lo)
            plsc.subcore_barrier()

            pltpu.sync_copy(acc.at[pl.ds(sid * frows, frows)],
                            OUT.at[pl.ds(lo + sid * frows, frows)])
            plsc.subcore_barrier()

    return kern


def _sc_edge3():
    per_tile = _EP // 32
    blocks = per_tile // _B
    mesh = plsc.VectorSubcoreMesh(core_axis_name="c", subcore_axis_name="s")

    @functools.partial(
        pl.kernel,
        out_type=[jax.ShapeDtypeStruct((_NP, 16), jnp.float32),
                  jax.ShapeDtypeStruct((_NP, 16), jnp.float32)],
        mesh=mesh,
        compiler_params=pltpu.CompilerParams(
            use_tc_tiling_on_sc=False, needs_layout_passes=False),
        scratch_types=[
            pltpu.VMEM((_B,), jnp.int32),
            pltpu.VMEM((_B,), jnp.int32),
            pltpu.VMEM((_B, 16), jnp.float32),
            pltpu.VMEM((_B, 16), jnp.float32),
            pltpu.VMEM((_B, 16), jnp.float32),
            pltpu.VMEM_SHARED((_NP, 16), jnp.float32),
            pltpu.SemaphoreType.DMA,
            pltpu.SemaphoreType.DMA,
        ],
    )
    def kern(T3, SRC, DST, O0, O1, s_idx, d_idx, rowS, rowD, bbuf, acc,
             sem1, sem2):
        cid = lax.axis_index("c")
        sid = lax.axis_index("s")
        lanes = lax.iota(jnp.int32, 16)
        zv = jnp.zeros((16,), jnp.float32)
        zi = jnp.zeros((16,), jnp.int32)
        base = (sid * 2 + cid) * per_tile

        def zrow(i, carry):
            bbuf[i, pl.ds(0, 16)] = zv
            return carry
        lax.fori_loop(0, _B, zrow, 0)

        for i in range(25):
            pltpu.sync_copy(bbuf, acc.at[pl.ds(sid * 3200 + i * _B, _B)])
        plsc.subcore_barrier()

        def do_block(b, carry):
            off = base + b * _B
            pltpu.sync_copy(SRC.at[pl.ds(off, _B)], s_idx)
            pltpu.sync_copy(DST.at[pl.ds(off, _B)], d_idx)
            cp1 = pltpu.async_copy(T3.at[s_idx], rowS, sem1)
            cp2 = pltpu.async_copy(T3.at[d_idx], rowD, sem2)
            cp1.wait()
            cp2.wait()

            def f_g(v, c2):
                r = v * 16 + lanes
                h16 = plsc.load_gather(rowS, [r, zi])
                as16 = plsc.load_gather(rowS, [r, zi + 1])
                ad16 = plsc.load_gather(rowD, [r, zi + 2])
                e16 = as16 + ad16
                g16 = jnp.exp(jnp.maximum(e16, 0.2 * e16))
                plsc.store_scatter(bbuf, [r, zi], g16)
                plsc.store_scatter(bbuf, [r, zi + 1], g16 * h16)
                return c2
            lax.fori_loop(0, _B // 16, f_g, 0)

            pltpu.sync_copy(bbuf, acc.at[d_idx], add=True)
            return carry

        lax.fori_loop(0, blocks, do_block, 0)
        plsc.subcore_barrier()

        @pl.when(cid == 0)
        def _():
            pltpu.sync_copy(acc.at[pl.ds(sid * 3200, 3200)],
                            O0.at[pl.ds(sid * 3200, 3200)])

        @pl.when(cid == 1)
        def _():
            pltpu.sync_copy(acc.at[pl.ds(sid * 3200, 3200)],
                            O1.at[pl.ds(sid * 3200, 3200)])

    return kern


def _proj_mat(a_src, a_dst, heads, out_ch, c):
    A = jnp.zeros((c, 8), jnp.float32)
    hr = jnp.arange(c) // out_ch
    cr = jnp.arange(c) % out_ch
    A = A.at[jnp.arange(c), hr].set(a_src[hr, cr])
    A = A.at[jnp.arange(c), 4 + hr].set(a_dst[hr, cr])
    return A


def _expand_mat(heads, out_ch):
    c = heads * out_ch
    E = jnp.zeros((4, c), jnp.float32)
    return E.at[jnp.arange(c) // out_ch, jnp.arange(c)].set(1.0)


def kernel(x, edge_index, W1, a_src1, a_dst1, b1, W2, a_src2, a_dst2, b2,
           W3, a_src3, a_dst3, b3):
    n = x.shape[0]
    loop = jnp.arange(n, dtype=edge_index.dtype)
    src = jnp.concatenate([edge_index[0], loop])
    dst = jnp.concatenate([edge_index[1], loop])
    srcp = jnp.pad(src, (0, _EP - src.shape[0]))
    dstp = jnp.pad(dst, (0, _EP - dst.shape[0]), constant_values=_NP - 1)

    xp = jnp.pad(x, ((0, _NP - n), (0, 1)))
    W1p = jnp.pad(W1, ((0, 1), (0, 0)))
    A1 = _proj_mat(a_src1, a_dst1, 4, 32, 128)
    A2 = _proj_mat(a_src2, a_dst2, 4, 16, 64)
    E1 = _expand_mat(4, 32)
    E2 = _expand_mat(4, 16)
    row3 = jnp.zeros((1, 16), jnp.float32)
    row3 = row3.at[0, 0].set(1.0).at[0, 1].set(a_src3[0, 0])
    row3 = row3.at[0, 2].set(a_dst3[0, 0])
    W3eff = W3 @ row3

    T1, AD1 = _tc_prep1(xp, W1p, A1)
    OUT1 = _sc_edge(128, 6400)(T1, AD1, srcp, dstp)
    T2, AD2 = _tc_mid(OUT1, E1, b1.reshape(1, 128), W2, A2, 128, 64)
    OUT2 = _sc_edge(64, 12800)(T2, AD2, srcp, dstp)
    T3 = _tc_fin2(OUT2, E2, b2.reshape(1, 64), W3eff)
    O0, O1 = _sc_edge3()(T3, srcp, dstp)
    z = _tc_final(O0, O1, b3.reshape(1, 1))
    return z[:_N, 0]


# speedup vs baseline: 52.0553x; 1.4451x over previous
import functools

import jax
import jax.numpy as jnp
from jax import lax
from jax.experimental import pallas as pl
from jax.experimental.pallas import tpu as pltpu
from jax.experimental.pallas import tpu_sc as plsc

_N = 50000
_NP = 51200
_CHUNK = 12800
_ACC_ROWS = 12816
_E = 800000
_EP = 851968
_RB = 1600


def _tc_prep1(xp, W1p, A1):
    def body(x_ref, w_ref, a_ref, t_ref, ad_ref):
        h = jnp.dot(x_ref[...], w_ref[...], preferred_element_type=jnp.float32)
        aa = jnp.dot(h, a_ref[...], preferred_element_type=jnp.float32)
        t_ref[:, 0:128] = h
        t_ref[:, 128:136] = aa
        t_ref[:, 136:144] = jnp.zeros((_RB, 8), jnp.float32)
        ad_ref[:, 0:4] = aa[:, 4:8]
        ad_ref[:, 4:16] = jnp.zeros((_RB, 12), jnp.float32)

    return pl.pallas_call(
        body,
        grid=(_NP // _RB,),
        in_specs=[
            pl.BlockSpec((_RB, 8), lambda i: (i, 0)),
            pl.BlockSpec((8, 128), lambda i: (0, 0)),
            pl.BlockSpec((128, 8), lambda i: (0, 0)),
        ],
        out_specs=[
            pl.BlockSpec((_RB, 144), lambda i: (i, 0)),
            pl.BlockSpec((_RB, 16), lambda i: (i, 0)),
        ],
        out_shape=[
            jax.ShapeDtypeStruct((_NP, 144), jnp.float32),
            jax.ShapeDtypeStruct((_NP, 16), jnp.float32),
        ],
    )(xp, W1p, A1)


def _tc_mid(OUT, E, b, W, A, C_in, C_out):
    RW_in = C_in + 16
    RW_out = C_out + 16

    def body(o_ref, e_ref, b_ref, w_ref, a_ref, t_ref, ad_ref):
        den = jnp.dot(o_ref[:, 0:4], e_ref[...],
                      preferred_element_type=jnp.float32)
        o = o_ref[:, 16:RW_in] / (den + 1e-16) + b_ref[...]
        o = jnp.where(o > 0, o, (jnp.exp(o) - 1.0))
        h = jnp.dot(o, w_ref[...], preferred_element_type=jnp.float32)
        aa = jnp.dot(h, a_ref[...], preferred_element_type=jnp.float32)
        t_ref[:, 0:C_out] = h
        t_ref[:, C_out:C_out + 8] = aa
        t_ref[:, C_out + 8:RW_out] = jnp.zeros((_RB, 8), jnp.float32)
        ad_ref[:, 0:4] = aa[:, 4:8]
        ad_ref[:, 4:16] = jnp.zeros((_RB, 12), jnp.float32)

    return pl.pallas_call(
        body,
        grid=(_NP // _RB,),
        in_specs=[
            pl.BlockSpec((_RB, RW_in), lambda i: (i, 0)),
            pl.BlockSpec((4, C_in), lambda i: (0, 0)),
            pl.BlockSpec((1, C_in), lambda i: (0, 0)),
            pl.BlockSpec((C_in, C_out), lambda i: (0, 0)),
            pl.BlockSpec((C_out, 8), lambda i: (0, 0)),
        ],
        out_specs=[
            pl.BlockSpec((_RB, RW_out), lambda i: (i, 0)),
            pl.BlockSpec((_RB, 16), lambda i: (i, 0)),
        ],
        out_shape=[
            jax.ShapeDtypeStruct((_NP, RW_out), jnp.float32),
            jax.ShapeDtypeStruct((_NP, 16), jnp.float32),
        ],
    )(OUT, E, b, W, A)


def _tc_fin2(OUT, E, b, W3eff):
    def body(o_ref, e_ref, b_ref, w_ref, t_ref):
        den = jnp.dot(o_ref[:, 0:4], e_ref[...],
                      preferred_element_type=jnp.float32)
        o = o_ref[:, 16:80] / (den + 1e-16) + b_ref[...]
        o = jnp.where(o > 0, o, (jnp.exp(o) - 1.0))
        t_ref[...] = jnp.dot(o, w_ref[...], preferred_element_type=jnp.float32)

    return pl.pallas_call(
        body,
        grid=(_NP // _RB,),
        in_specs=[
            pl.BlockSpec((_RB, 80), lambda i: (i, 0)),
            pl.BlockSpec((4, 64), lambda i: (0, 0)),
            pl.BlockSpec((1, 64), lambda i: (0, 0)),
            pl.BlockSpec((64, 16), lambda i: (0, 0)),
        ],
        out_specs=pl.BlockSpec((_RB, 16), lambda i: (i, 0)),
        out_shape=jax.ShapeDtypeStruct((_NP, 16), jnp.float32),
    )(OUT, E, b, W3eff)


def _tc_final(O0, O1, b3s):
    RB = 2048

    def body(o0_ref, o1_ref, b_ref, z_ref):
        acc = o0_ref[...] + o1_ref[...]
        z = acc[:, 1:2] / (acc[:, 0:1] + 1e-16) + b_ref[...]
        z_ref[...] = jax.nn.sigmoid(z) * 100.0

    return pl.pallas_call(
        body,
        grid=(_NP // RB,),
        in_specs=[
            pl.BlockSpec((RB, 16), lambda i: (i, 0)),
            pl.BlockSpec((RB, 16), lambda i: (i, 0)),
            pl.BlockSpec((1, 1), lambda i: (0, 0)),
        ],
        out_specs=pl.BlockSpec((RB, 1), lambda i: (i, 0)),
        out_shape=jax.ShapeDtypeStruct((_NP, 1), jnp.float32),
    )(O0, O1, b3s)


_B = 128


def _sc_edge(C, chunk):
    RW = C + 16
    acc_rows = chunk + 16
    zrows = acc_rows // 16
    frows = chunk // 16
    cps = _NP // chunk // 2
    blocks = _EP // 16 // _B
    mesh = plsc.VectorSubcoreMesh(core_axis_name="c", subcore_axis_name="s")

    @functools.partial(
        pl.kernel,
        out_type=jax.ShapeDtypeStruct((_NP, RW), jnp.float32),
        mesh=mesh,
        compiler_params=pltpu.CompilerParams(
            use_tc_tiling_on_sc=False, needs_layout_passes=False),
        scratch_types=[
            pltpu.VMEM((_B,), jnp.int32),
            pltpu.VMEM((_B,), jnp.int32),
            pltpu.VMEM((_B,), jnp.int32),
            pltpu.VMEM((2 * _B,), jnp.int32),
            pltpu.VMEM((2 * _B,), jnp.int32),
            pltpu.VMEM((_B,), jnp.int32),
            pltpu.VMEM((_B,), jnp.int32),
            pltpu.VMEM((_B, 16), jnp.float32),
            pltpu.VMEM((_B, RW), jnp.float32),
            pltpu.VMEM((_B, RW), jnp.float32),
            pltpu.VMEM((4 * _B,), jnp.float32),
            pltpu.VMEM_SHARED((acc_rows, RW), jnp.float32),
            pltpu.SemaphoreType.DMA,
            pltpu.SemaphoreType.DMA,
        ],
    )
    def kern(T, AD, SRC, DST, OUT, s_idx, d_idx, l_idx, csrc, cdst, gsrc,
             gdst, adbuf, rowbuf, bbuf, gbuf, acc, sem1, sem2):
        cid = lax.axis_index("c")
        sid = lax.axis_index("s")
        lanes = lax.iota(jnp.int32, 16)
        zv = jnp.zeros((16,), jnp.float32)
        ebase = sid * (_EP // 16)

        def zero_bbuf():
            def zrow(i, carry):
                r = i // (RW // 16)
                co = (i % (RW // 16)) * 16
                bbuf[r, pl.ds(co, 16)] = zv
                return carry
            lax.fori_loop(0, _B * (RW // 16), zrow, 0)

        def process_batch(lo):
            def f_l(v, carry):
                d16 = gdst[pl.ds(v * 16, 16)]
                l16 = jnp.minimum(jnp.maximum(d16 - lo, 0), chunk)
                l_idx[pl.ds(v * 16, 16)] = l16
                return carry
            lax.fori_loop(0, _B // 16, f_l, 0)

            cp1 = pltpu.async_copy(T.at[gsrc], rowbuf, sem1)
            cp2 = pltpu.async_copy(AD.at[gdst], adbuf, sem2)
            cp1.wait()
            cp2.wait()

            @functools.partial(plsc.parallel_loop, 0, _B // 4, unroll=2)
            def f_g(v):
                e4 = v * 4 + lanes // 4
                hh = lanes % 4
                as16 = plsc.load_gather(rowbuf, [e4, C + hh])
                ad16 = plsc.load_gather(adbuf, [e4, hh])
                e16 = as16 + ad16
                g16 = jnp.exp(jnp.maximum(e16, 0.2 * e16))
                plsc.store_scatter(gbuf, [e4 * 4 + hh], g16)

            zi = jnp.zeros((16,), jnp.int32)

            @functools.partial(plsc.parallel_loop, 0, _B, unroll=2)
            def f_b(e):
                g4 = plsc.load_gather(gbuf, [e * 4 + jnp.minimum(lanes, 3)])
                plsc.store_scatter(bbuf, [zi + e, lanes], g4, mask=lanes < 4)
                for j in range(C // 16):
                    head = (j * 16) // (C // 4)
                    ge = plsc.load_gather(gbuf, [zi + (e * 4 + head)])
                    h16 = rowbuf[e, pl.ds(j * 16, 16)]
                    bbuf[e, pl.ds(16 + j * 16, 16)] = h16 * ge

            pltpu.sync_copy(bbuf, acc.at[l_idx], add=True)

        def scan_block(b, cnt, lo):
            off = ebase + b * _B
            pltpu.sync_copy(SRC.at[pl.ds(off, _B)], s_idx)
            pltpu.sync_copy(DST.at[pl.ds(off, _B)], d_idx)

            def f_app(v, c2):
                d16 = d_idx[pl.ds(v * 16, 16)]
                l16 = d16 - lo
                ok = (l16 >= 0) & (l16 < chunk)
                s16 = s_idx[pl.ds(v * 16, 16)]
                plsc.store_compressed(csrc.at[pl.ds(c2, 16)], s16, mask=ok)
                plsc.store_compressed(cdst.at[pl.ds(c2, 16)], d16, mask=ok)
                return c2 + jnp.sum(ok.astype(jnp.int32))
            cnt = lax.fori_loop(0, _B // 16, f_app, cnt)

            @pl.when(cnt >= _B)
            def _():
                def f_cp(v, c3):
                    gsrc[pl.ds(v * 16, 16)] = csrc[pl.ds(v * 16, 16)]
                    gdst[pl.ds(v * 16, 16)] = cdst[pl.ds(v * 16, 16)]
                    return c3
                lax.fori_loop(0, _B // 16, f_cp, 0)
                process_batch(lo)
                def f_sh(v, c3):
                    csrc[pl.ds(v * 16, 16)] = csrc[pl.ds(_B + v * 16, 16)]
                    cdst[pl.ds(v * 16, 16)] = cdst[pl.ds(_B + v * 16, 16)]
                    return c3
                lax.fori_loop(0, _B // 16, f_sh, 0)
            return jnp.where(cnt >= _B, cnt - _B, cnt)

        def flush_tail(cnt, lo):
            def f_t(v, carry):
                li = v * 16 + lanes
                m = li < cnt
                s16 = csrc[pl.ds(v * 16, 16)]
                d16 = cdst[pl.ds(v * 16, 16)]
                gsrc[pl.ds(v * 16, 16)] = jnp.where(m, s16, 0)
                gdst[pl.ds(v * 16, 16)] = jnp.where(m, d16, _NP - 1)
                return carry
            lax.fori_loop(0, _B // 16, f_t, 0)
            process_batch(lo)

        for k_ in range(cps):
            lo = (cid * cps + k_) * chunk
            zero_bbuf()
            for i in range(zrows // _B):
                pltpu.sync_copy(bbuf, acc.at[pl.ds(sid * zrows + i * _B, _B)])
            pltpu.sync_copy(bbuf.at[pl.ds(0, zrows % _B)],
                            acc.at[pl.ds(sid * zrows + (zrows // _B) * _B,
                                         zrows % _B)])
            plsc.subcore_barrier()

            def f_blk(b, cnt):
                return scan_block(b, cnt, lo)
            cnt = lax.fori_loop(0, blocks, f_blk, jnp.int32(0))
            flush_tail(cnt, lo)
            plsc.subcore_barrier()

            pltpu.sync_copy(acc.at[pl.ds(sid * frows, frows)],
                            OUT.at[pl.ds(lo + sid * frows, frows)])
            plsc.subcore_barrier()

    return kern


def _sc_edge3():
    per_tile = _EP // 32
    blocks = per_tile // _B
    mesh = plsc.VectorSubcoreMesh(core_axis_name="c", subcore_axis_name="s")

    @functools.partial(
        pl.kernel,
        out_type=[jax.ShapeDtypeStruct((_NP, 16), jnp.float32),
                  jax.ShapeDtypeStruct((_NP, 16), jnp.float32)],
        mesh=mesh,
        compiler_params=pltpu.CompilerParams(
            use_tc_tiling_on_sc=False, needs_layout_passes=False),
        scratch_types=[
            pltpu.VMEM((_B,), jnp.int32),
            pltpu.VMEM((_B,), jnp.int32),
            pltpu.VMEM((_B, 16), jnp.float32),
            pltpu.VMEM((_B, 16), jnp.float32),
            pltpu.VMEM((_B, 16), jnp.float32),
            pltpu.VMEM_SHARED((_NP, 16), jnp.float32),
            pltpu.SemaphoreType.DMA,
            pltpu.SemaphoreType.DMA,
        ],
    )
    def kern(T3, SRC, DST, O0, O1, s_idx, d_idx, rowS, rowD, bbuf, acc,
             sem1, sem2):
        cid = lax.axis_index("c")
        sid = lax.axis_index("s")
        lanes = lax.iota(jnp.int32, 16)
        zv = jnp.zeros((16,), jnp.float32)
        zi = jnp.zeros((16,), jnp.int32)
        base = (sid * 2 + cid) * per_tile

        def zrow(i, carry):
            bbuf[i, pl.ds(0, 16)] = zv
            return carry
        lax.fori_loop(0, _B, zrow, 0)

        for i in range(25):
            pltpu.sync_copy(bbuf, acc.at[pl.ds(sid * 3200 + i * _B, _B)])
        plsc.subcore_barrier()

        def do_block(b, carry):
            off = base + b * _B
            pltpu.sync_copy(SRC.at[pl.ds(off, _B)], s_idx)
            pltpu.sync_copy(DST.at[pl.ds(off, _B)], d_idx)
            cp1 = pltpu.async_copy(T3.at[s_idx], rowS, sem1)
            cp2 = pltpu.async_copy(T3.at[d_idx], rowD, sem2)
            cp1.wait()
            cp2.wait()

            def f_g(v, c2):
                r = v * 16 + lanes
                h16 = plsc.load_gather(rowS, [r, zi])
                as16 = plsc.load_gather(rowS, [r, zi + 1])
                ad16 = plsc.load_gather(rowD, [r, zi + 2])
                e16 = as16 + ad16
                g16 = jnp.exp(jnp.maximum(e16, 0.2 * e16))
                plsc.store_scatter(bbuf, [r, zi], g16)
                plsc.store_scatter(bbuf, [r, zi + 1], g16 * h16)
                return c2
            lax.fori_loop(0, _B // 16, f_g, 0)

            pltpu.sync_copy(bbuf, acc.at[d_idx], add=True)
            return carry

        lax.fori_loop(0, blocks, do_block, 0)
        plsc.subcore_barrier()

        @pl.when(cid == 0)
        def _():
            pltpu.sync_copy(acc.at[pl.ds(sid * 3200, 3200)],
                            O0.at[pl.ds(sid * 3200, 3200)])

        @pl.when(cid == 1)
        def _():
            pltpu.sync_copy(acc.at[pl.ds(sid * 3200, 3200)],
                            O1.at[pl.ds(sid * 3200, 3200)])

    return kern


def _proj_mat(a_src, a_dst, heads, out_ch, c):
    A = jnp.zeros((c, 8), jnp.float32)
    hr = jnp.arange(c) // out_ch
    cr = jnp.arange(c) % out_ch
    A = A.at[jnp.arange(c), hr].set(a_src[hr, cr])
    A = A.at[jnp.arange(c), 4 + hr].set(a_dst[hr, cr])
    return A


def _expand_mat(heads, out_ch):
    c = heads * out_ch
    E = jnp.zeros((4, c), jnp.float32)
    return E.at[jnp.arange(c) // out_ch, jnp.arange(c)].set(1.0)


def kernel(x, edge_index, W1, a_src1, a_dst1, b1, W2, a_src2, a_dst2, b2,
           W3, a_src3, a_dst3, b3):
    n = x.shape[0]
    loop = jnp.arange(n, dtype=edge_index.dtype)
    src = jnp.concatenate([edge_index[0], loop])
    dst = jnp.concatenate([edge_index[1], loop])
    srcp = jnp.pad(src, (0, _EP - src.shape[0]))
    dstp = jnp.pad(dst, (0, _EP - dst.shape[0]), constant_values=_NP - 1)

    xp = jnp.pad(x, ((0, _NP - n), (0, 1)))
    W1p = jnp.pad(W1, ((0, 1), (0, 0)))
    A1 = _proj_mat(a_src1, a_dst1, 4, 32, 128)
    A2 = _proj_mat(a_src2, a_dst2, 4, 16, 64)
    E1 = _expand_mat(4, 32)
    E2 = _expand_mat(4, 16)
    row3 = jnp.zeros((1, 16), jnp.float32)
    row3 = row3.at[0, 0].set(1.0).at[0, 1].set(a_src3[0, 0])
    row3 = row3.at[0, 2].set(a_dst3[0, 0])
    W3eff = W3 @ row3

    T1, AD1 = _tc_prep1(xp, W1p, A1)
    OUT1 = _sc_edge(128, 6400)(T1, AD1, srcp, dstp)
    T2, AD2 = _tc_mid(OUT1, E1, b1.reshape(1, 128), W2, A2, 128, 64)
    OUT2 = _sc_edge(64, 12800)(T2, AD2, srcp, dstp)
    T3 = _tc_fin2(OUT2, E2, b2.reshape(1, 64), W3eff)
    O0, O1 = _sc_edge3()(T3, srcp, dstp)
    z = _tc_final(O0, O1, b3.reshape(1, 1))
    return z[:_N, 0]
